# Initial kernel scaffold; baseline (speedup 1.0000x reference)
import jax
import jax.numpy as jnp
from jax.experimental import pallas as pl


def kernel(x, Wq, Wk):
    raise NotImplementedError("write your pallas kernel here")


# speedup vs baseline: 46.3481x; 46.3481x over previous
import jax
import jax.numpy as jnp
from jax import lax
from jax.experimental import pallas as pl
from jax.experimental.pallas import tpu as pltpu

STRIDE_K = 4
SPARSE_KEEP = 0.5
IDX_DIM = 32
NEG_INF = float("-inf")


def _ks_body(xs_ref, wk_ref, ks_ref):
    ks_ref[0] = lax.dot_general(
        xs_ref[0], wk_ref[...], (((1,), (1,)), ((), ())),
        preferred_element_type=jnp.float32)


def _main_body(x_ref, wq_ref, ks_ref, out_ref, e_scratch):
    R = x_ref.shape[1]
    P = ks_ref.shape[1]
    S = out_ref.shape[2]
    b = pl.program_id(0)
    i = pl.program_id(1)

    @pl.when((b == 0) & (i == 0))
    def _():
        jj = lax.broadcasted_iota(jnp.int32, (P, S), 0)
        cc = lax.broadcasted_iota(jnp.int32, (P, S), 1)
        e_scratch[...] = (cc == jj * STRIDE_K).astype(jnp.float32)

    q = lax.dot_general(
        x_ref[0], wq_ref[...], (((1,), (1,)), ((), ())),
        preferred_element_type=jnp.float32)
    s = lax.dot_general(
        q, ks_ref[0], (((1,), (1,)), ((), ())),
        preferred_element_type=jnp.float32)
    s = jnp.maximum(s, jnp.float32(0.0))

    s_int = lax.bitcast_convert_type(s, jnp.int32) & jnp.int32(0x7FFFFFFF)

    rows = i * R + lax.broadcasted_iota(jnp.int32, (R, 1), 0)
    n = rows // STRIDE_K + 1
    k = jnp.maximum(1, n // 2)
    j_idx = lax.broadcasted_iota(jnp.int32, (R, P), 1)
    valid = j_idx < n
    s_int = jnp.where(valid, s_int, jnp.int32(-1))

    lo = jnp.zeros((R, 1), jnp.int32)
    hi = jnp.max(s_int, axis=1, keepdims=True)

    def bs_body(_, carry):
        lo, hi = carry
        d = hi - lo
        mid = lo + (d >> 1) + (d & 1)
        c = jnp.sum((s_int >= mid).astype(jnp.int32), axis=1, keepdims=True)
        pred = c >= k
        return jnp.where(pred, mid, lo), jnp.where(pred, hi, mid - 1)

    lo, hi = lax.fori_loop(0, 31, bs_body, (lo, hi))
    t = lo

    gt = s_int > t
    eq = s_int == t
    c_gt = jnp.sum(gt.astype(jnp.int32), axis=1, keepdims=True)
    rem = (k - c_gt).astype(jnp.float32)

    jj = lax.broadcasted_iota(jnp.int32, (P, P), 0)
    ii = lax.broadcasted_iota(jnp.int32, (P, P), 1)
    lt_mat = (jj < ii).astype(jnp.float32)
    cum_ex = lax.dot_general(
        eq.astype(jnp.float32), lt_mat, (((1,), (0,)), ((), ())),
        preferred_element_type=jnp.float32)

    sel = gt | (eq & (cum_ex < rem))
    sel_f = sel.astype(jnp.float32)

    marker = lax.dot_general(
        sel_f, e_scratch[...], (((1,), (0,)), ((), ())),
        preferred_element_type=jnp.float32)
    out_ref[0] = jnp.where(marker > jnp.float32(0.5), jnp.float32(0.0),
                           NEG_INF)


def kernel(x, Wq, Wk):
    B, S, D = x.shape
    P = (S - 1) // STRIDE_K + 1
    R = 256

    xs = x[:, ::STRIDE_K, :]

    ks = pl.pallas_call(
        _ks_body,
        grid=(B,),
        in_specs=[
            pl.BlockSpec((1, P, D), lambda b: (b, 0, 0)),
            pl.BlockSpec((IDX_DIM, D), lambda b: (0, 0)),
        ],
        out_specs=pl.BlockSpec((1, P, IDX_DIM), lambda b: (b, 0, 0)),
        out_shape=jax.ShapeDtypeStruct((B, P, IDX_DIM), jnp.float32),
    )(xs, Wk)

    full = pl.pallas_call(
        _main_body,
        grid=(B, S // R),
        in_specs=[
            pl.BlockSpec((1, R, D), lambda b, i: (b, i, 0)),
            pl.BlockSpec((IDX_DIM, D), lambda b, i: (0, 0)),
            pl.BlockSpec((1, P, IDX_DIM), lambda b, i: (b, 0, 0)),
        ],
        out_specs=pl.BlockSpec((1, R, S), lambda b, i: (b, i, 0)),
        out_shape=jax.ShapeDtypeStruct((B, S, S), jnp.float32),
        scratch_shapes=[pltpu.VMEM((P, S), jnp.float32)],
    )(x, Wq, ks)

    return full[:, None, :, :]
